# baseline (speedup 1.0000x reference)
import jax
import jax.numpy as jnp
from jax.experimental import pallas as pl
from jax.experimental.pallas import tpu as pltpu

_TM = 512


def _gate_kernel(x_ref, w_ref, o_ref):
    i = pl.program_id(0)
    o_ref[pl.ds(i * _TM, _TM), :] = jax.lax.dot_general(
        x_ref[...],
        w_ref[...],
        dimension_numbers=(((1,), (1,)), ((), ())),
        preferred_element_type=jnp.float32,
    )


def kernel(x, W_gate):
    t, d = x.shape
    e = W_gate.shape[0]
    return pl.pallas_call(
        _gate_kernel,
        grid=(t // _TM,),
        in_specs=[
            pl.BlockSpec((_TM, d), lambda i: (i, 0)),
            pl.BlockSpec((e, d), lambda i: (0, 0)),
        ],
        out_specs=pl.BlockSpec((t, e), lambda i: (0, 0)),
        out_shape=jax.ShapeDtypeStruct((t, e), jnp.float32),
        compiler_params=pltpu.CompilerParams(
            dimension_semantics=(pltpu.ARBITRARY,),
        ),
    )(x, W_gate)
